# baseline (speedup 1.0000x reference)
import functools

import jax
import jax.numpy as jnp
from jax import lax
from jax.experimental import pallas as pl
from jax.experimental.pallas import tpu as pltpu
from jax.experimental.pallas import tpu_sc as plsc

N_ROW = 10000
N_COL = 10000
D = 128
N_LAYERS = 3

NC = 2
NS = 16
NW = NC * NS
L = 16
CHUNK = 32
QCH = 40
ROWS_PER_TILE = 624
FULL = ROWS_PER_TILE // CHUNK
REM = ROWS_PER_TILE % CHUNK
TAIL = N_ROW - NS * ROWS_PER_TILE


_GATHER_DNUMS = lax.GatherDimensionNumbers(
    offset_dims=(), collapsed_slice_dims=(0,), start_index_map=(0,))


def _bcast_lane(vec, lane):
    idx = jnp.full((L, 1), lane, jnp.int32)
    return lax.gather(vec, idx, _GATHER_DNUMS, (1,),
                      mode=lax.GatherScatterMode.PROMISE_IN_BOUNDS)


NBUF = 8
GA = 6


def _sc_agg_body(table, gidx, sidx, wts, out,
                 acc, gbuf, sbuf, wbuf,
                 a0, a1, a2, a3, a4, a5, a6, a7,
                 semg0, semg1, semg2, semg3, semg4, semg5, semg6, semg7,
                 sems0, sems1, sems2, sems3, sems4, sems5, sems6, sems7):
    cpt = gidx.shape[0] // NW
    c = lax.axis_index("c")
    s = lax.axis_index("s")
    wid = c * NS + s
    abufs = (a0, a1, a2, a3, a4, a5, a6, a7)
    semgs = (semg0, semg1, semg2, semg3, semg4, semg5, semg6, semg7)
    semss = (sems0, sems1, sems2, sems3, sems4, sems5, sems6, sems7)

    def zrow(r, carry):
        for j in range(D // L):
            a0[r, pl.ds(j * L, L)] = jnp.zeros((L,), jnp.float32)
        return carry

    lax.fori_loop(0, CHUNK, zrow, 0)
    base = s * ROWS_PER_TILE
    for k in range(FULL):
        pltpu.sync_copy(a0, acc.at[pl.ds(base + k * CHUNK, CHUNK)])
    if REM:
        pltpu.sync_copy(a0.at[pl.ds(0, REM)],
                        acc.at[pl.ds(base + FULL * CHUNK, REM)])

    @pl.when(s == NS - 1)
    def _zero_tail():
        pltpu.sync_copy(a0.at[pl.ds(0, TAIL)],
                        acc.at[pl.ds(NS * ROWS_PER_TILE, TAIL)])

    def stage(q, carry):
        qoff = wid * cpt + q * QCH
        pltpu.sync_copy(gidx.at[pl.ds(qoff, QCH)], gbuf)
        pltpu.sync_copy(sidx.at[pl.ds(qoff, QCH)], sbuf)
        pltpu.sync_copy(wts.at[pl.ds(qoff, QCH)], wbuf)
        for b in range(GA):
            pltpu.async_copy(table.at[gbuf.at[b]], abufs[b], semgs[b])

        def chunk_quad(p4, inner):
            for b in range(NBUF):
                p = NBUF * p4 + b
                ab = abufs[b]
                ng = (b + GA) % NBUF

                @pl.when(p >= 2)
                def _wait_prev_scatter():
                    pltpu.make_async_copy(abufs[ng], acc.at[sbuf.at[p]],
                                          semss[ng]).wait()

                @pl.when(p + GA < QCH)
                def _issue_next():
                    pltpu.async_copy(table.at[gbuf.at[p + GA]], abufs[ng],
                                     semgs[ng])

                pltpu.make_async_copy(table.at[gbuf.at[p]], ab, semgs[b]).wait()

                def grp_body(t, ii):
                    w16 = wbuf[p, pl.ds(t * L, L)]
                    for lane in range(L):
                        wv = _bcast_lane(w16, lane)
                        r = t * L + lane
                        for j in range(D // L):
                            sl = pl.ds(j * L, L)
                            ab[r, sl] = ab[r, sl] * wv
                    return ii

                lax.fori_loop(0, CHUNK // L, grp_body, 0)
                pltpu.async_copy(ab, acc.at[sbuf.at[p]], semss[b], add=True)
            return inner

        lax.fori_loop(0, QCH // NBUF, chunk_quad, 0)
        for b in range(2):
            p = QCH - 2 + b
            pltpu.make_async_copy(abufs[p % NBUF], acc.at[sbuf.at[p]],
                                  semss[p % NBUF]).wait()
        return carry

    lax.fori_loop(0, cpt // QCH, stage, 0)
    plsc.subcore_barrier()

    ob = c * N_ROW + base
    for k in range(FULL):
        pltpu.sync_copy(acc.at[pl.ds(base + k * CHUNK, CHUNK)],
                        out.at[pl.ds(ob + k * CHUNK, CHUNK)])
    if REM:
        pltpu.sync_copy(acc.at[pl.ds(base + FULL * CHUNK, REM)],
                        out.at[pl.ds(ob + FULL * CHUNK, REM)])

    @pl.when(s == NS - 1)
    def _drain_tail():
        pltpu.sync_copy(acc.at[pl.ds(NS * ROWS_PER_TILE, TAIL)],
                        out.at[pl.ds(c * N_ROW + NS * ROWS_PER_TILE, TAIL)])


def _make_sc_agg(num_idx_rows):
    mesh = plsc.VectorSubcoreMesh(core_axis_name="c", subcore_axis_name="s",
                                  num_cores=NC, num_subcores=NS)
    cpt = num_idx_rows // NW
    return pl.kernel(
        _sc_agg_body,
        out_type=jax.ShapeDtypeStruct((NC * N_ROW, D), jnp.float32),
        mesh=mesh,
        scratch_types=[
            pltpu.VMEM_SHARED((N_ROW, D), jnp.float32),
            pltpu.VMEM((QCH, CHUNK), jnp.int32),
            pltpu.VMEM((QCH, CHUNK), jnp.int32),
            pltpu.VMEM((QCH, CHUNK), jnp.float32),
        ] + [pltpu.VMEM((CHUNK, D), jnp.float32)] * NBUF
          + [pltpu.SemaphoreType.DMA] * (2 * NBUF),
    )


_BLK = 2000
_NB = N_ROW // _BLK


def _prep_body(xr_ref, wr_ref, br_ref, xc_ref, wc_ref, bc_ref, ew_ref,
               hr_ref, hc_ref, w_ref):
    hr_ref[:] = (
        jnp.dot(xr_ref[:], wr_ref[:], preferred_element_type=jnp.float32,
                precision=lax.Precision.HIGHEST) + br_ref[:])
    hc_ref[:] = (
        jnp.dot(xc_ref[:], wc_ref[:], preferred_element_type=jnp.float32,
                precision=lax.Precision.HIGHEST) + bc_ref[:])
    w_ref[:] = jax.nn.sigmoid(ew_ref[:])


def _prep(xr, wr, br, xc, wc, bc, ew2d):
    return pl.pallas_call(
        _prep_body,
        out_shape=[
            jax.ShapeDtypeStruct((xr.shape[0], D), jnp.float32),
            jax.ShapeDtypeStruct((xc.shape[0], D), jnp.float32),
            jax.ShapeDtypeStruct(ew2d.shape, jnp.float32),
        ],
    )(xr, wr, br.reshape(1, D), xc, wc, bc.reshape(1, D), ew2d)


def _leaky(x):
    return jnp.where(x >= 0, x, 0.01 * x)


def _mlp_body(eps_ref, h_ref, p0_ref, p1_ref, w0_ref, b0_ref, w1_ref, b1_ref,
              y_ref, yres_ref):
    eps = eps_ref[0, 0]
    x = (1.0 + eps) * h_ref[:] + p0_ref[:] + p1_ref[:]
    x = _leaky(jnp.dot(x, w0_ref[:], preferred_element_type=jnp.float32,
                       precision=lax.Precision.HIGHEST) + b0_ref[:])
    x = _leaky(jnp.dot(x, w1_ref[:], preferred_element_type=jnp.float32,
                       precision=lax.Precision.HIGHEST) + b1_ref[:])
    y_ref[:] = x
    yres_ref[:] = x + h_ref[:]


def _mlp(eps, h, parts, w0, b0, w1, b1):
    return pl.pallas_call(
        _mlp_body,
        grid=(_NB,),
        in_specs=[
            pl.BlockSpec((1, 1), lambda i: (0, 0), memory_space=pltpu.SMEM),
            pl.BlockSpec((_BLK, D), lambda i: (i, 0)),
            pl.BlockSpec((_BLK, D), lambda i: (i, 0)),
            pl.BlockSpec((_BLK, D), lambda i: (i + _NB, 0)),
            pl.BlockSpec((D, D), lambda i: (0, 0)),
            pl.BlockSpec((1, D), lambda i: (0, 0)),
            pl.BlockSpec((D, D), lambda i: (0, 0)),
            pl.BlockSpec((1, D), lambda i: (0, 0)),
        ],
        out_specs=[
            pl.BlockSpec((_BLK, D), lambda i: (i, 0)),
            pl.BlockSpec((_BLK, D), lambda i: (i, 0)),
        ],
        out_shape=[
            jax.ShapeDtypeStruct((N_ROW, D), jnp.float32),
            jax.ShapeDtypeStruct((N_ROW, D), jnp.float32),
        ],
    )(eps.reshape(1, 1), h, parts, parts, w0, b0.reshape(1, D), w1,
      b1.reshape(1, D))


def kernel(row_feat, col_feat, edge_weight, edge_src_col, edge_dst_row,
           W_in_row, b_in_row, W_in_col, b_in_col,
           eps_c2r, eps_r2c, W_c2r, b_c2r, W_r2c, b_r2c):
    e = edge_weight.shape[0]
    e_pad = -(-e // (NW * CHUNK * 8)) * (NW * CHUNK * 8)
    pad = e_pad - e

    h_row, h_col, w = _prep(row_feat, W_in_row, b_in_row,
                            col_feat, W_in_col, b_in_col,
                            edge_weight.reshape(-1, CHUNK))
    w = w.reshape(-1)

    pad_idx = jnp.arange(pad, dtype=jnp.int32) % jnp.int32(N_ROW)
    src = jnp.concatenate([edge_src_col, pad_idx]).reshape(-1, CHUNK)
    dst = jnp.concatenate([edge_dst_row, pad_idx]).reshape(-1, CHUNK)
    wp = jnp.concatenate([w, jnp.zeros((pad,), jnp.float32)]).reshape(-1, CHUNK)

    sc_agg = _make_sc_agg(src.shape[0])

    for i in range(N_LAYERS):
        parts = sc_agg(h_col, src, dst, wp)
        hr_mlp, hr_res = _mlp(eps_c2r[i], h_row, parts,
                              W_c2r[i, 0], b_c2r[i, 0], W_c2r[i, 1], b_c2r[i, 1])
        parts = sc_agg(hr_mlp, dst, src, wp)
        _, hc_res = _mlp(eps_r2c[i], h_col, parts,
                         W_r2c[i, 0], b_r2c[i, 0], W_r2c[i, 1], b_r2c[i, 1])
        h_row, h_col = hr_res, hc_res

    return jnp.concatenate([h_row, h_col], axis=0)


# speedup vs baseline: 1.2064x; 1.2064x over previous
import jax
import jax.numpy as jnp
from jax import lax
from jax.experimental import pallas as pl
from jax.experimental.pallas import tpu as pltpu
from jax.experimental.pallas import tpu_sc as plsc

N_ROW = 10000
N_COL = 10000
D = 128
N_LAYERS = 3

NC = 2
NS = 16
NW = NC * NS
L = 16
CHUNK = 64
QCH = 40
ROWS_PER_TILE = 624
FULL = ROWS_PER_TILE // CHUNK
REM = ROWS_PER_TILE % CHUNK
TAIL = N_ROW - NS * ROWS_PER_TILE


_GATHER_DNUMS = lax.GatherDimensionNumbers(
    offset_dims=(), collapsed_slice_dims=(0,), start_index_map=(0,))


def _bcast_lane(vec, lane):
    idx = jnp.full((L, 1), lane, jnp.int32)
    return lax.gather(vec, idx, _GATHER_DNUMS, (1,),
                      mode=lax.GatherScatterMode.PROMISE_IN_BOUNDS)


NBUF = 4
GA = 2


def _sc_agg_body(table, gidx, sidx, wts, out,
                 acc, gbuf, sbuf, wbuf, a0, a1, a2, a3,
                 semg0, semg1, semg2, semg3,
                 sems0, sems1, sems2, sems3):
    cpt = gidx.shape[0] // NW
    c = lax.axis_index("c")
    s = lax.axis_index("s")
    wid = c * NS + s
    abufs = (a0, a1, a2, a3)
    semgs = (semg0, semg1, semg2, semg3)
    semss = (sems0, sems1, sems2, sems3)

    def zrow(r, carry):
        for j in range(D // L):
            a0[r, pl.ds(j * L, L)] = jnp.zeros((L,), jnp.float32)
        return carry

    lax.fori_loop(0, CHUNK, zrow, 0)
    base = s * ROWS_PER_TILE
    for k in range(FULL):
        pltpu.sync_copy(a0, acc.at[pl.ds(base + k * CHUNK, CHUNK)])
    if REM:
        pltpu.sync_copy(a0.at[pl.ds(0, REM)],
                        acc.at[pl.ds(base + FULL * CHUNK, REM)])

    @pl.when(s == NS - 1)
    def _zero_tail():
        pltpu.sync_copy(a0.at[pl.ds(0, TAIL)],
                        acc.at[pl.ds(NS * ROWS_PER_TILE, TAIL)])

    def stage(q, carry):
        qoff = wid * cpt + q * QCH
        pltpu.sync_copy(gidx.at[pl.ds(qoff, QCH)], gbuf)
        pltpu.sync_copy(sidx.at[pl.ds(qoff, QCH)], sbuf)
        pltpu.sync_copy(wts.at[pl.ds(qoff, QCH)], wbuf)
        for b in range(GA):
            pltpu.async_copy(table.at[gbuf.at[b]], abufs[b], semgs[b])

        def chunk_quad(p4, inner):
            for b in range(NBUF):
                p = NBUF * p4 + b
                ab = abufs[b]
                ng = (b + GA) % NBUF

                @pl.when(p >= 2)
                def _wait_prev_scatter():
                    pltpu.make_async_copy(abufs[ng], acc.at[sbuf.at[p]],
                                          semss[ng]).wait()

                @pl.when(p + GA < QCH)
                def _issue_next():
                    pltpu.async_copy(table.at[gbuf.at[p + GA]], abufs[ng],
                                     semgs[ng])

                pltpu.make_async_copy(table.at[gbuf.at[p]], ab, semgs[b]).wait()

                def grp_body(t, ii):
                    w16 = wbuf[p, pl.ds(t * L, L)]
                    for lane in range(L):
                        wv = _bcast_lane(w16, lane)
                        r = t * L + lane
                        for j in range(D // L):
                            sl = pl.ds(j * L, L)
                            ab[r, sl] = ab[r, sl] * wv
                    return ii

                lax.fori_loop(0, CHUNK // L, grp_body, 0)
                pltpu.async_copy(ab, acc.at[sbuf.at[p]], semss[b], add=True)
            return inner

        lax.fori_loop(0, QCH // NBUF, chunk_quad, 0)
        for b in range(2):
            p = QCH - 2 + b
            pltpu.make_async_copy(abufs[p % NBUF], acc.at[sbuf.at[p]],
                                  semss[p % NBUF]).wait()
        return carry

    lax.fori_loop(0, cpt // QCH, stage, 0)
    plsc.subcore_barrier()

    ob = c * N_ROW + base
    for k in range(FULL):
        pltpu.sync_copy(acc.at[pl.ds(base + k * CHUNK, CHUNK)],
                        out.at[pl.ds(ob + k * CHUNK, CHUNK)])
    if REM:
        pltpu.sync_copy(acc.at[pl.ds(base + FULL * CHUNK, REM)],
                        out.at[pl.ds(ob + FULL * CHUNK, REM)])

    @pl.when(s == NS - 1)
    def _drain_tail():
        pltpu.sync_copy(acc.at[pl.ds(NS * ROWS_PER_TILE, TAIL)],
                        out.at[pl.ds(c * N_ROW + NS * ROWS_PER_TILE, TAIL)])


def _make_sc_agg(num_idx_rows):
    mesh = plsc.VectorSubcoreMesh(core_axis_name="c", subcore_axis_name="s",
                                  num_cores=NC, num_subcores=NS)
    cpt = num_idx_rows // NW
    return pl.kernel(
        _sc_agg_body,
        out_type=jax.ShapeDtypeStruct((NC * N_ROW, D), jnp.float32),
        mesh=mesh,
        scratch_types=[
            pltpu.VMEM_SHARED((N_ROW, D), jnp.float32),
            pltpu.VMEM((QCH, CHUNK), jnp.int32),
            pltpu.VMEM((QCH, CHUNK), jnp.int32),
            pltpu.VMEM((QCH, CHUNK), jnp.float32),
        ] + [pltpu.VMEM((CHUNK, D), jnp.float32)] * NBUF
          + [pltpu.SemaphoreType.DMA] * (2 * NBUF),
    )


_BLK = 2000
_NB = N_ROW // _BLK


def _prep_body(xr_ref, wr_ref, br_ref, xc_ref, wc_ref, bc_ref, ew_ref,
               hr_ref, hc_ref, w_ref):
    hr_ref[:] = (
        jnp.dot(xr_ref[:], wr_ref[:], preferred_element_type=jnp.float32) + br_ref[:])
    hc_ref[:] = (
        jnp.dot(xc_ref[:], wc_ref[:],
                preferred_element_type=jnp.float32) + bc_ref[:])
    w_ref[:] = jax.nn.sigmoid(ew_ref[:])


def _prep(xr, wr, br, xc, wc, bc, ew2d):
    return pl.pallas_call(
        _prep_body,
        out_shape=[
            jax.ShapeDtypeStruct((xr.shape[0], D), jnp.float32),
            jax.ShapeDtypeStruct((xc.shape[0], D), jnp.float32),
            jax.ShapeDtypeStruct(ew2d.shape, jnp.float32),
        ],
    )(xr, wr, br.reshape(1, D), xc, wc, bc.reshape(1, D), ew2d)


def _leaky(x):
    return jnp.where(x >= 0, x, 0.01 * x)


def _mlp_body(eps_ref, h_ref, p0_ref, p1_ref, w0_ref, b0_ref, w1_ref, b1_ref,
              y_ref, yres_ref):
    eps = eps_ref[0, 0]
    x = (1.0 + eps) * h_ref[:] + p0_ref[:] + p1_ref[:]
    x = _leaky(jnp.dot(x, w0_ref[:],
                       preferred_element_type=jnp.float32) + b0_ref[:])
    x = _leaky(jnp.dot(x, w1_ref[:],
                       preferred_element_type=jnp.float32) + b1_ref[:])
    y_ref[:] = x
    yres_ref[:] = x + h_ref[:]


def _mlp(eps, h, parts, w0, b0, w1, b1):
    return pl.pallas_call(
        _mlp_body,
        grid=(_NB,),
        in_specs=[
            pl.BlockSpec((1, 1), lambda i: (0, 0), memory_space=pltpu.SMEM),
            pl.BlockSpec((_BLK, D), lambda i: (i, 0)),
            pl.BlockSpec((_BLK, D), lambda i: (i, 0)),
            pl.BlockSpec((_BLK, D), lambda i: (i + _NB, 0)),
            pl.BlockSpec((D, D), lambda i: (0, 0)),
            pl.BlockSpec((1, D), lambda i: (0, 0)),
            pl.BlockSpec((D, D), lambda i: (0, 0)),
            pl.BlockSpec((1, D), lambda i: (0, 0)),
        ],
        out_specs=[
            pl.BlockSpec((_BLK, D), lambda i: (i, 0)),
            pl.BlockSpec((_BLK, D), lambda i: (i, 0)),
        ],
        out_shape=[
            jax.ShapeDtypeStruct((N_ROW, D), jnp.float32),
            jax.ShapeDtypeStruct((N_ROW, D), jnp.float32),
        ],
    )(eps.reshape(1, 1), h, parts, parts, w0, b0.reshape(1, D), w1,
      b1.reshape(1, D))


def kernel(row_feat, col_feat, edge_weight, edge_src_col, edge_dst_row,
           W_in_row, b_in_row, W_in_col, b_in_col,
           eps_c2r, eps_r2c, W_c2r, b_c2r, W_r2c, b_r2c):
    e = edge_weight.shape[0]
    e_pad = -(-e // (NW * CHUNK * 8)) * (NW * CHUNK * 8)
    pad = e_pad - e

    h_row, h_col, w = _prep(row_feat, W_in_row, b_in_row,
                            col_feat, W_in_col, b_in_col,
                            edge_weight.reshape(-1, CHUNK))
    w = w.reshape(-1)

    pad_idx = jnp.arange(pad, dtype=jnp.int32) % jnp.int32(N_ROW)
    src = jnp.concatenate([edge_src_col, pad_idx]).reshape(-1, CHUNK)
    dst = jnp.concatenate([edge_dst_row, pad_idx]).reshape(-1, CHUNK)
    wp = jnp.concatenate([w, jnp.zeros((pad,), jnp.float32)]).reshape(-1, CHUNK)

    sc_agg = _make_sc_agg(src.shape[0])

    for i in range(N_LAYERS):
        parts = sc_agg(h_col, src, dst, wp)
        hr_mlp, hr_res = _mlp(eps_c2r[i], h_row, parts,
                              W_c2r[i, 0], b_c2r[i, 0], W_c2r[i, 1], b_c2r[i, 1])
        parts = sc_agg(hr_mlp, dst, src, wp)
        _, hc_res = _mlp(eps_r2c[i], h_col, parts,
                         W_r2c[i, 0], b_r2c[i, 0], W_r2c[i, 1], b_r2c[i, 1])
        h_row, h_col = hr_res, hc_res

    return jnp.concatenate([h_row, h_col], axis=0)
